# Initial kernel scaffold; baseline (speedup 1.0000x reference)
import jax
import jax.numpy as jnp
from jax.experimental import pallas as pl


def kernel(x, weight):
    raise NotImplementedError("write your pallas kernel here")


# speedup vs baseline: 1.6231x; 1.6231x over previous
import functools

import jax
import jax.numpy as jnp
from jax import lax
from jax.experimental import pallas as pl
from jax.experimental.pallas import tpu as pltpu
from jax.experimental.pallas import tpu_sc as plsc

MAX_LEN = 8192
HIDDEN = 2048
BATCH = 4
T_LEN = 4096
B_TOTAL = BATCH * T_LEN

_NC = 2
_NS = 16
_NW = _NC * _NS
_BPW = B_TOTAL // _NW
_C = 32
_NCH = _BPW // _C


def _make_gather():
    mesh = plsc.VectorSubcoreMesh(core_axis_name="c", subcore_axis_name="s")

    @functools.partial(
        pl.kernel,
        mesh=mesh,
        out_type=jax.ShapeDtypeStruct((B_TOTAL, HIDDEN), jnp.float32),
        scratch_types=[
            pltpu.VMEM((_BPW,), jnp.int32),
            pltpu.VMEM((_C, HIDDEN), jnp.float32),
            pltpu.SemaphoreType.DMA,
        ],
    )
    def gather_kernel(idx_hbm, table_hbm, out_hbm, idx_v, rows_v, sem):
        wid = lax.axis_index("s") * _NC + lax.axis_index("c")
        base = wid * _BPW
        pltpu.sync_copy(idx_hbm.at[pl.ds(base, _BPW)], idx_v)

        def chunk_body(g, carry):
            pltpu.async_copy(
                table_hbm.at[idx_v.at[pl.ds(g * _C, _C)]], rows_v, sem
            ).wait()
            pltpu.sync_copy(rows_v, out_hbm.at[pl.ds(base + g * _C, _C)])
            return carry

        lax.fori_loop(0, _NCH, chunk_body, 0)

    return gather_kernel


_gather = _make_gather()


def kernel(x, weight):
    batch_size, t_length = x.shape
    idx = x.reshape(-1).astype(jnp.int32)
    out = _gather(idx, weight)
    return out.reshape(batch_size, t_length, HIDDEN)


# speedup vs baseline: 1.7832x; 1.0986x over previous
import functools

import jax
import jax.numpy as jnp
from jax import lax
from jax.experimental import pallas as pl
from jax.experimental.pallas import tpu as pltpu
from jax.experimental.pallas import tpu_sc as plsc

MAX_LEN = 8192
HIDDEN = 2048
BATCH = 4
T_LEN = 4096
B_TOTAL = BATCH * T_LEN

_NC = 2
_NS = 16
_NW = _NC * _NS
_BPW = B_TOTAL // _NW
_C = 16
_NCH = _BPW // _C


def _make_gather():
    mesh = plsc.VectorSubcoreMesh(core_axis_name="c", subcore_axis_name="s")

    @functools.partial(
        pl.kernel,
        mesh=mesh,
        out_type=jax.ShapeDtypeStruct((B_TOTAL, HIDDEN), jnp.float32),
        scratch_types=[
            pltpu.VMEM((_BPW,), jnp.int32),
            pltpu.VMEM((2, _C, HIDDEN), jnp.float32),
            pltpu.SemaphoreType.DMA,
            pltpu.SemaphoreType.DMA,
            pltpu.SemaphoreType.DMA,
            pltpu.SemaphoreType.DMA,
        ],
    )
    def gather_kernel(idx_hbm, table_hbm, out_hbm, idx_v, rows_v,
                      gsem0, gsem1, ssem0, ssem1):
        wid = lax.axis_index("s") * _NC + lax.axis_index("c")
        base = wid * _BPW
        pltpu.sync_copy(idx_hbm.at[pl.ds(base, _BPW)], idx_v)
        gsems = (gsem0, gsem1)
        ssems = (ssem0, ssem1)

        def g_src(g):
            return table_hbm.at[idx_v.at[pl.ds(g * _C, _C)]]

        def o_dst(g):
            return out_hbm.at[pl.ds(base + g * _C, _C)]

        for b in range(2):
            pltpu.async_copy(g_src(b), rows_v.at[b], gsems[b])

        def outer(j, carry):
            for b in range(2):
                g = 2 * j + b
                pltpu.make_async_copy(g_src(g), rows_v.at[b], gsems[b]).wait()
                pltpu.async_copy(rows_v.at[b], o_dst(g), ssems[b])
                pltpu.make_async_copy(rows_v.at[b], o_dst(g), ssems[b]).wait()
                pltpu.async_copy(g_src(g + 2), rows_v.at[b], gsems[b])
            return carry

        lax.fori_loop(0, _NCH // 2 - 1, outer, 0)

        for b in range(2):
            g = _NCH - 2 + b
            pltpu.make_async_copy(g_src(g), rows_v.at[b], gsems[b]).wait()
            pltpu.async_copy(rows_v.at[b], o_dst(g), ssems[b])
        for b in range(2):
            g = _NCH - 2 + b
            pltpu.make_async_copy(rows_v.at[b], o_dst(g), ssems[b]).wait()

    return gather_kernel


_gather = _make_gather()


def kernel(x, weight):
    batch_size, t_length = x.shape
    idx = x.reshape(-1).astype(jnp.int32)
    out = _gather(idx, weight)
    return out.reshape(batch_size, t_length, HIDDEN)
